# Initial kernel scaffold; baseline (speedup 1.0000x reference)
import jax
import jax.numpy as jnp
from jax.experimental import pallas as pl


def kernel(u, v, e_indices, e_values, Wg, bg, Wf, bf, Wt, bt):
    raise NotImplementedError("write your pallas kernel here")


# speedup vs baseline: 2.6492x; 2.6492x over previous
import functools

import jax
import jax.numpy as jnp
from jax import lax
from jax.experimental import pallas as pl
from jax.experimental.pallas import tpu as pltpu
from jax.experimental.pallas import tpu_sc as plsc

U = 10000
V = 10000
E = 320000
F = 128
H = 16
D = 64

NC = 2
NS = 16
NW = NC * NS
EW = E // NW
K = 80
NCHUNK = EW // K
NZT = 10
RPT = U // NZT
ZR = 200


def _proj_uv_body(u_ref, v_ref, wu_ref, wv_ref, bg_ref, a_ref, b_ref):
    a_ref[...] = (
        jnp.dot(u_ref[...].astype(jnp.bfloat16), wu_ref[...].astype(jnp.bfloat16),
                preferred_element_type=jnp.float32)
        + bg_ref[...]
    )
    b_ref[...] = jnp.dot(v_ref[...].astype(jnp.bfloat16), wv_ref[...].astype(jnp.bfloat16),
                         preferred_element_type=jnp.float32)


def _tc_proj_uv(u, v, wu, wv, bg2):
    bu = 1000
    return pl.pallas_call(
        _proj_uv_body,
        grid=(U // bu,),
        in_specs=[
            pl.BlockSpec((bu, F), lambda i: (i, 0)),
            pl.BlockSpec((bu, F), lambda i: (i, 0)),
            pl.BlockSpec((F, D), lambda i: (0, 0)),
            pl.BlockSpec((F, D), lambda i: (0, 0)),
            pl.BlockSpec((1, D), lambda i: (0, 0)),
        ],
        out_specs=[
            pl.BlockSpec((bu, D), lambda i: (i, 0)),
            pl.BlockSpec((bu, D), lambda i: (i, 0)),
        ],
        out_shape=[
            jax.ShapeDtypeStruct((U, D), jnp.float32),
            jax.ShapeDtypeStruct((V, D), jnp.float32),
        ],
    )(u, v, wu, wv, bg2)


def _proj_e_body(e_ref, we_ref, c_ref):
    c_ref[...] = jnp.dot(e_ref[...].astype(jnp.bfloat16), we_ref[...].astype(jnp.bfloat16),
                         preferred_element_type=jnp.float32)


def _tc_proj_e(e_values, we):
    be = 8000
    return pl.pallas_call(
        _proj_e_body,
        grid=(E // be,),
        in_specs=[
            pl.BlockSpec((be, H), lambda i: (i, 0)),
            pl.BlockSpec((H, D), lambda i: (0, 0)),
        ],
        out_specs=pl.BlockSpec((be, D), lambda i: (i, 0)),
        out_shape=jax.ShapeDtypeStruct((E, D), jnp.float32),
    )(e_values, we)


def _sc_agg(a, b, c, iu, iv):
    mesh = plsc.VectorSubcoreMesh(
        core_axis_name="c", subcore_axis_name="s", num_cores=NC, num_subcores=NS
    )

    @functools.partial(
        pl.kernel,
        out_type=(
            jax.ShapeDtypeStruct((U, D), jnp.float32),
            jax.ShapeDtypeStruct((U, D), jnp.float32),
        ),
        mesh=mesh,
        scratch_types=[
            pltpu.VMEM((K,), jnp.int32),
            pltpu.VMEM((K,), jnp.int32),
            pltpu.VMEM((K, D), jnp.float32),
            pltpu.VMEM((K, D), jnp.float32),
            pltpu.VMEM((K, D), jnp.float32),
            pltpu.VMEM((ZR, D), jnp.float32),
            pltpu.VMEM_SHARED((U, D), jnp.float32),
            pltpu.SemaphoreType.DMA,
            pltpu.SemaphoreType.DMA,
        ],
        compiler_params=pltpu.CompilerParams(use_tc_tiling_on_sc=False),
    )
    def body(a_hbm, b_hbm, c_hbm, iu_hbm, iv_hbm, out0, out1,
             iu_v, iv_v, ar, br, cr, zr, agg, sem_a, sem_b):
        cid = lax.axis_index("c")
        sid = lax.axis_index("s")
        wid = cid * NS + sid
        r0 = sid * RPT

        @pl.when(sid < NZT)
        def _():
            @pl.loop(0, ZR)
            def _(r):
                for q in range(D // 16):
                    zr[r, pl.ds(q * 16, 16)] = jnp.zeros((16,), jnp.float32)

            for t in range(RPT // ZR):
                pltpu.sync_copy(zr, agg.at[pl.ds(r0 + t * ZR, ZR)])

        plsc.subcore_barrier()

        @pl.loop(0, NCHUNK)
        def _(i):
            base = wid * EW + i * K
            pltpu.sync_copy(iu_hbm.at[pl.ds(base, K)], iu_v)
            pltpu.sync_copy(iv_hbm.at[pl.ds(base, K)], iv_v)
            cpa = pltpu.async_copy(a_hbm.at[iu_v], ar, sem_a)
            cpb = pltpu.async_copy(b_hbm.at[iv_v], br, sem_b)
            pltpu.sync_copy(c_hbm.at[pl.ds(base, K)], cr)
            cpa.wait()
            cpb.wait()

            @pl.loop(0, K, unroll=4)
            def _(r):
                for q in range(D // 16):
                    sl = pl.ds(q * 16, 16)
                    cr[r, sl] = jnp.maximum(ar[r, sl] + br[r, sl] + cr[r, sl], 0.0)

            pltpu.sync_copy(cr, agg.at[iu_v], add=True)

        plsc.subcore_barrier()

        @pl.when(jnp.logical_and(cid == 0, sid < NZT))
        def _():
            pltpu.sync_copy(agg.at[pl.ds(r0, RPT)], out0.at[pl.ds(r0, RPT)])

        @pl.when(jnp.logical_and(cid == 1, sid < NZT))
        def _():
            pltpu.sync_copy(agg.at[pl.ds(r0, RPT)], out1.at[pl.ds(r0, RPT)])

    return body(a, b, c, iu, iv)


def _final_body(u_ref, p0_ref, p1_ref, wfu_ref, wfa_ref, bf_ref, wt_ref,
                bt_ref, o_ref):
    agg = p0_ref[...] + p1_ref[...]
    f = (
        jnp.dot(u_ref[...].astype(jnp.bfloat16), wfu_ref[...].astype(jnp.bfloat16),
                preferred_element_type=jnp.float32)
        + jnp.dot(agg.astype(jnp.bfloat16), wfa_ref[...].astype(jnp.bfloat16),
                  preferred_element_type=jnp.float32)
        + bf_ref[...]
    )
    f = jnp.maximum(f, 0.0)
    fb = f.astype(jnp.bfloat16).astype(jnp.float32)
    wtb = wt_ref[...].astype(jnp.bfloat16).astype(jnp.float32)
    t = jnp.sum(fb * wtb, axis=1, keepdims=True) + bt_ref[...]
    o_ref[...] = 1.0 / (1.0 + jnp.exp(-t))


def _tc_final(u, p0, p1, wfu, wfa, bf2, wt2, bt2):
    bu = 1000
    return pl.pallas_call(
        _final_body,
        grid=(U // bu,),
        in_specs=[
            pl.BlockSpec((bu, F), lambda i: (i, 0)),
            pl.BlockSpec((bu, D), lambda i: (i, 0)),
            pl.BlockSpec((bu, D), lambda i: (i, 0)),
            pl.BlockSpec((F, D), lambda i: (0, 0)),
            pl.BlockSpec((D, D), lambda i: (0, 0)),
            pl.BlockSpec((1, D), lambda i: (0, 0)),
            pl.BlockSpec((1, D), lambda i: (0, 0)),
            pl.BlockSpec((1, 1), lambda i: (0, 0)),
        ],
        out_specs=pl.BlockSpec((bu, 1), lambda i: (i, 0)),
        out_shape=jax.ShapeDtypeStruct((U, 1), jnp.float32),
    )(u, p0, p1, wfu, wfa, bf2, wt2, bt2)


def kernel(u, v, e_indices, e_values, Wg, bg, Wf, bf, Wt, bt):
    iv = e_indices[0].astype(jnp.int32)
    iu = e_indices[1].astype(jnp.int32)
    a, b = _tc_proj_uv(u, v, Wg[:F], Wg[F:2 * F], bg.reshape(1, D))
    c = _tc_proj_e(e_values, Wg[2 * F:])
    p0, p1 = _sc_agg(a, b, c, iu, iv)
    return _tc_final(u, p0, p1, Wf[:F], Wf[F:], bf.reshape(1, D),
                     Wt.reshape(1, D), bt.reshape(1, 1))


# speedup vs baseline: 3.5553x; 1.3420x over previous
import functools

import jax
import jax.numpy as jnp
from jax import lax
from jax.experimental import pallas as pl
from jax.experimental.pallas import tpu as pltpu
from jax.experimental.pallas import tpu_sc as plsc

U = 10000
V = 10000
E = 320000
F = 128
H = 16
D = 64

NC = 2
NS = 16
NW = NC * NS
EW = E // NW
K = 80
NCHUNK = EW // K
NZT = 10
RPT = U // NZT
ZR = 200


def _proj_uv_body(u_ref, v_ref, wu_ref, wv_ref, bg_ref, a_ref, b_ref):
    a_ref[...] = (
        jnp.dot(u_ref[...].astype(jnp.bfloat16), wu_ref[...].astype(jnp.bfloat16),
                preferred_element_type=jnp.float32)
        + bg_ref[...]
    )
    b_ref[...] = jnp.dot(v_ref[...].astype(jnp.bfloat16), wv_ref[...].astype(jnp.bfloat16),
                         preferred_element_type=jnp.float32)


def _tc_proj_uv(u, v, wu, wv, bg2):
    bu = 1000
    return pl.pallas_call(
        _proj_uv_body,
        grid=(U // bu,),
        in_specs=[
            pl.BlockSpec((bu, F), lambda i: (i, 0)),
            pl.BlockSpec((bu, F), lambda i: (i, 0)),
            pl.BlockSpec((F, D), lambda i: (0, 0)),
            pl.BlockSpec((F, D), lambda i: (0, 0)),
            pl.BlockSpec((1, D), lambda i: (0, 0)),
        ],
        out_specs=[
            pl.BlockSpec((bu, D), lambda i: (i, 0)),
            pl.BlockSpec((bu, D), lambda i: (i, 0)),
        ],
        out_shape=[
            jax.ShapeDtypeStruct((U, D), jnp.float32),
            jax.ShapeDtypeStruct((V, D), jnp.float32),
        ],
    )(u, v, wu, wv, bg2)


def _proj_e_body(e_ref, we_ref, c_ref):
    c_ref[...] = jnp.dot(e_ref[...].astype(jnp.bfloat16), we_ref[...].astype(jnp.bfloat16),
                         preferred_element_type=jnp.float32)


def _tc_proj_e(e_values, we):
    be = 8000
    return pl.pallas_call(
        _proj_e_body,
        grid=(E // be,),
        in_specs=[
            pl.BlockSpec((be, H), lambda i: (i, 0)),
            pl.BlockSpec((H, D), lambda i: (0, 0)),
        ],
        out_specs=pl.BlockSpec((be, D), lambda i: (i, 0)),
        out_shape=jax.ShapeDtypeStruct((E, D), jnp.float32),
    )(e_values, we)


def _sc_agg(a, b, c, iu, iv):
    mesh = plsc.VectorSubcoreMesh(
        core_axis_name="c", subcore_axis_name="s", num_cores=NC, num_subcores=NS
    )

    @functools.partial(
        pl.kernel,
        out_type=(
            jax.ShapeDtypeStruct((U, D), jnp.float32),
            jax.ShapeDtypeStruct((U, D), jnp.float32),
        ),
        mesh=mesh,
        scratch_types=[
            pltpu.VMEM((NCHUNK, K), jnp.int32),
            pltpu.VMEM((NCHUNK, K), jnp.int32),
            pltpu.VMEM((K, D), jnp.float32),
            pltpu.VMEM((K, D), jnp.float32),
            pltpu.VMEM((K, D), jnp.float32),
            pltpu.VMEM((K, D), jnp.float32),
            pltpu.VMEM((K, D), jnp.float32),
            pltpu.VMEM((K, D), jnp.float32),
            pltpu.VMEM((ZR, D), jnp.float32),
            pltpu.VMEM_SHARED((U, D), jnp.float32),
            pltpu.SemaphoreType.DMA,
            pltpu.SemaphoreType.DMA,
            pltpu.SemaphoreType.DMA,
            pltpu.SemaphoreType.DMA,
            pltpu.SemaphoreType.DMA,
            pltpu.SemaphoreType.DMA,
            pltpu.SemaphoreType.DMA,
            pltpu.SemaphoreType.DMA,
        ],
        compiler_params=pltpu.CompilerParams(use_tc_tiling_on_sc=False),
    )
    def body(a_hbm, b_hbm, c_hbm, iu_hbm, iv_hbm, out0, out1,
             iu2, iv2, ar0, br0, cr0, ar1, br1, cr1, zr, agg,
             sa0, sb0, sc0, sa1, sb1, sc1, ss0, ss1):
        cid = lax.axis_index("c")
        sid = lax.axis_index("s")
        wid = cid * NS + sid
        r0 = sid * RPT
        bufs = ((ar0, br0, cr0, sa0, sb0, sc0, ss0),
                (ar1, br1, cr1, sa1, sb1, sc1, ss1))

        pltpu.sync_copy(iu_hbm.at[wid], iu2)
        pltpu.sync_copy(iv_hbm.at[wid], iv2)

        @pl.when(sid < NZT)
        def _():
            @pl.loop(0, ZR)
            def _(r):
                for q in range(D // 16):
                    zr[r, pl.ds(q * 16, 16)] = jnp.zeros((16,), jnp.float32)

            for t in range(RPT // ZR):
                pltpu.sync_copy(zr, agg.at[pl.ds(r0 + t * ZR, ZR)])

        plsc.subcore_barrier()

        def issue(ci, p):
            ar, br, cr, sa, sb, sc, _ = bufs[p]
            pltpu.async_copy(a_hbm.at[iu2.at[ci]], ar, sa)
            pltpu.async_copy(b_hbm.at[iv2.at[ci]], br, sb)
            pltpu.async_copy(c_hbm.at[pl.ds(wid * EW + ci * K, K)], cr, sc)

        def wait_in(ci, p):
            ar, br, cr, sa, sb, sc, _ = bufs[p]
            pltpu.make_async_copy(a_hbm.at[iu2.at[ci]], ar, sa).wait()
            pltpu.make_async_copy(b_hbm.at[iv2.at[ci]], br, sb).wait()
            pltpu.make_async_copy(c_hbm.at[pl.ds(wid * EW + ci * K, K)], cr, sc).wait()

        def compute_scatter(ci, p):
            ar, br, cr, _, _, _, ss = bufs[p]

            @pl.loop(0, K, unroll=4)
            def _(r):
                for q in range(D // 16):
                    sl = pl.ds(q * 16, 16)
                    cr[r, sl] = jnp.maximum(ar[r, sl] + br[r, sl] + cr[r, sl], 0.0)

            pltpu.async_copy(cr, agg.at[iu2.at[ci]], ss, add=True)

        def wait_scatter(ci, p):
            cr, ss = bufs[p][2], bufs[p][6]
            pltpu.make_async_copy(cr, agg.at[iu2.at[ci]], ss).wait()

        issue(0, 0)

        @pl.loop(0, (NCHUNK - 1) // 2)
        def _(j):
            for p in range(2):
                ci = 2 * j + p
                wait_in(ci, p)
                @pl.when(ci > 0)
                def _():
                    wait_scatter(ci - 1, 1 - p)
                issue(ci + 1, 1 - p)
                compute_scatter(ci, p)

        ci_last = NCHUNK - 1
        wait_in(ci_last, ci_last % 2)
        wait_scatter(ci_last - 1, 1 - ci_last % 2)
        compute_scatter(ci_last, ci_last % 2)
        wait_scatter(ci_last, ci_last % 2)

        plsc.subcore_barrier()

        @pl.when(jnp.logical_and(cid == 0, sid < NZT))
        def _():
            pltpu.sync_copy(agg.at[pl.ds(r0, RPT)], out0.at[pl.ds(r0, RPT)])

        @pl.when(jnp.logical_and(cid == 1, sid < NZT))
        def _():
            pltpu.sync_copy(agg.at[pl.ds(r0, RPT)], out1.at[pl.ds(r0, RPT)])

    return body(a, b, c, iu.reshape(NW, NCHUNK, K), iv.reshape(NW, NCHUNK, K))


def _final_body(u_ref, p0_ref, p1_ref, wfu_ref, wfa_ref, bf_ref, wt_ref,
                bt_ref, o_ref):
    agg = p0_ref[...] + p1_ref[...]
    f = (
        jnp.dot(u_ref[...].astype(jnp.bfloat16), wfu_ref[...].astype(jnp.bfloat16),
                preferred_element_type=jnp.float32)
        + jnp.dot(agg.astype(jnp.bfloat16), wfa_ref[...].astype(jnp.bfloat16),
                  preferred_element_type=jnp.float32)
        + bf_ref[...]
    )
    f = jnp.maximum(f, 0.0)
    fb = f.astype(jnp.bfloat16).astype(jnp.float32)
    wtb = wt_ref[...].astype(jnp.bfloat16).astype(jnp.float32)
    t = jnp.sum(fb * wtb, axis=1, keepdims=True) + bt_ref[...]
    o_ref[...] = 1.0 / (1.0 + jnp.exp(-t))


def _tc_final(u, p0, p1, wfu, wfa, bf2, wt2, bt2):
    bu = 1000
    return pl.pallas_call(
        _final_body,
        grid=(U // bu,),
        in_specs=[
            pl.BlockSpec((bu, F), lambda i: (i, 0)),
            pl.BlockSpec((bu, D), lambda i: (i, 0)),
            pl.BlockSpec((bu, D), lambda i: (i, 0)),
            pl.BlockSpec((F, D), lambda i: (0, 0)),
            pl.BlockSpec((D, D), lambda i: (0, 0)),
            pl.BlockSpec((1, D), lambda i: (0, 0)),
            pl.BlockSpec((1, D), lambda i: (0, 0)),
            pl.BlockSpec((1, 1), lambda i: (0, 0)),
        ],
        out_specs=pl.BlockSpec((bu, 1), lambda i: (i, 0)),
        out_shape=jax.ShapeDtypeStruct((U, 1), jnp.float32),
    )(u, p0, p1, wfu, wfa, bf2, wt2, bt2)


def kernel(u, v, e_indices, e_values, Wg, bg, Wf, bf, Wt, bt):
    iv = e_indices[0].astype(jnp.int32)
    iu = e_indices[1].astype(jnp.int32)
    a, b = _tc_proj_uv(u, v, Wg[:F], Wg[F:2 * F], bg.reshape(1, D))
    c = _tc_proj_e(e_values, Wg[2 * F:])
    p0, p1 = _sc_agg(a, b, c, iu, iv)
    return _tc_final(u, p0, p1, Wf[:F], Wf[F:], bf.reshape(1, D),
                     Wt.reshape(1, D), bt.reshape(1, 1))


# speedup vs baseline: 4.8471x; 1.3634x over previous
import functools

import jax
import jax.numpy as jnp
from jax import lax
from jax.experimental import pallas as pl
from jax.experimental.pallas import tpu as pltpu
from jax.experimental.pallas import tpu_sc as plsc

U = 10000
V = 10000
E = 320000
F = 128
H = 16
D = 64

NC = 2
NS = 16
NW = NC * NS
EW = E // NW
K = 80
K2 = K // 2
EW2 = EW // 2
NCHUNK = EW // K
NZT = 10
RPT = U // NZT
ZR = 200


def _proj_uv_body(u_ref, v_ref, wu_ref, wv_ref, bg_ref, a_ref, b_ref):
    a_ref[...] = (
        jnp.dot(u_ref[...].astype(jnp.bfloat16), wu_ref[...].astype(jnp.bfloat16),
                preferred_element_type=jnp.float32)
        + bg_ref[...]
    )
    b_ref[...] = jnp.dot(v_ref[...].astype(jnp.bfloat16), wv_ref[...].astype(jnp.bfloat16),
                         preferred_element_type=jnp.float32)


def _tc_proj_uv(u, v, wu, wv, bg2):
    bu = 1000
    return pl.pallas_call(
        _proj_uv_body,
        grid=(U // bu,),
        in_specs=[
            pl.BlockSpec((bu, F), lambda i: (i, 0)),
            pl.BlockSpec((bu, F), lambda i: (i, 0)),
            pl.BlockSpec((F, D), lambda i: (0, 0)),
            pl.BlockSpec((F, D), lambda i: (0, 0)),
            pl.BlockSpec((1, D), lambda i: (0, 0)),
        ],
        out_specs=[
            pl.BlockSpec((bu, D), lambda i: (i, 0)),
            pl.BlockSpec((bu, D), lambda i: (i, 0)),
        ],
        out_shape=[
            jax.ShapeDtypeStruct((U, D), jnp.float32),
            jax.ShapeDtypeStruct((V, D), jnp.float32),
        ],
    )(u, v, wu, wv, bg2)


def _proj_e_body(e_ref, we_ref, c_ref):
    m = jnp.dot(e_ref[...].astype(jnp.bfloat16), we_ref[...],
                preferred_element_type=jnp.float32)
    c_ref[...] = m.reshape(m.shape[0] * 4, 2 * D)


def _tc_proj_e(e_values, we):
    pk = 8
    w8 = jax.scipy.linalg.block_diag(*([we.astype(jnp.bfloat16)] * pk))
    be = 4000
    return pl.pallas_call(
        _proj_e_body,
        grid=(E // pk // be,),
        in_specs=[
            pl.BlockSpec((be, pk * H), lambda i: (i, 0)),
            pl.BlockSpec((pk * H, pk * D), lambda i: (0, 0)),
        ],
        out_specs=pl.BlockSpec((4 * be, 2 * D), lambda i: (i, 0)),
        out_shape=jax.ShapeDtypeStruct((E // 2, 2 * D), jnp.float32),
    )(e_values.reshape(E // pk, pk * H), w8)


def _sc_agg(a, b, c, iu, iv):
    mesh = plsc.VectorSubcoreMesh(
        core_axis_name="c", subcore_axis_name="s", num_cores=NC, num_subcores=NS
    )

    @functools.partial(
        pl.kernel,
        out_type=(
            jax.ShapeDtypeStruct((U, D), jnp.float32),
            jax.ShapeDtypeStruct((U, D), jnp.float32),
        ),
        mesh=mesh,
        scratch_types=[
            pltpu.VMEM((NCHUNK, K), jnp.int32),
            pltpu.VMEM((NCHUNK, K), jnp.int32),
            pltpu.VMEM((K, D), jnp.float32),
            pltpu.VMEM((K, D), jnp.float32),
            pltpu.VMEM((K2, 2 * D), jnp.float32),
            pltpu.VMEM((K, D), jnp.float32),
            pltpu.VMEM((K, D), jnp.float32),
            pltpu.VMEM((K2, 2 * D), jnp.float32),
            pltpu.VMEM((ZR, D), jnp.float32),
            pltpu.VMEM_SHARED((U, D), jnp.float32),
            pltpu.SemaphoreType.DMA,
            pltpu.SemaphoreType.DMA,
            pltpu.SemaphoreType.DMA,
            pltpu.SemaphoreType.DMA,
            pltpu.SemaphoreType.DMA,
            pltpu.SemaphoreType.DMA,
            pltpu.SemaphoreType.DMA,
            pltpu.SemaphoreType.DMA,
        ],
        compiler_params=pltpu.CompilerParams(use_tc_tiling_on_sc=False),
    )
    def body(a_hbm, b_hbm, c_hbm, iu_hbm, iv_hbm, out0, out1,
             iu2, iv2, ar0, br0, cr0, ar1, br1, cr1, zr, agg,
             sa0, sb0, sc0, sa1, sb1, sc1, ss0, ss1):
        cid = lax.axis_index("c")
        sid = lax.axis_index("s")
        wid = cid * NS + sid
        r0 = sid * RPT
        bufs = ((ar0, br0, cr0, sa0, sb0, sc0, ss0),
                (ar1, br1, cr1, sa1, sb1, sc1, ss1))

        pltpu.sync_copy(iu_hbm.at[wid], iu2)
        pltpu.sync_copy(iv_hbm.at[wid], iv2)

        @pl.when(sid < NZT)
        def _():
            @pl.loop(0, ZR)
            def _(r):
                for q in range(D // 16):
                    zr[r, pl.ds(q * 16, 16)] = jnp.zeros((16,), jnp.float32)

            for t in range(RPT // ZR):
                pltpu.sync_copy(zr, agg.at[pl.ds(r0 + t * ZR, ZR)])

        plsc.subcore_barrier()

        def issue(ci, p):
            ar, br, cr, sa, sb, sc, _ = bufs[p]
            pltpu.async_copy(a_hbm.at[iu2.at[ci]], ar, sa)
            pltpu.async_copy(b_hbm.at[iv2.at[ci]], br, sb)
            pltpu.async_copy(c_hbm.at[pl.ds(wid * EW2 + ci * K2, K2)], cr, sc)

        def wait_in(ci, p):
            ar, br, cr, sa, sb, sc, _ = bufs[p]
            pltpu.make_async_copy(a_hbm.at[iu2.at[ci]], ar, sa).wait()
            pltpu.make_async_copy(b_hbm.at[iv2.at[ci]], br, sb).wait()
            pltpu.make_async_copy(c_hbm.at[pl.ds(wid * EW2 + ci * K2, K2)], cr, sc).wait()

        def compute_scatter(ci, p):
            ar, br, cr, _, _, _, ss = bufs[p]

            @pl.loop(0, K2, unroll=4)
            def _(rr):
                for p2 in range(2):
                    r = 2 * rr + p2
                    for q in range(D // 16):
                        sl = pl.ds(q * 16, 16)
                        ar[r, sl] = jnp.maximum(
                            ar[r, sl] + br[r, sl]
                            + cr[rr, pl.ds(p2 * D + q * 16, 16)], 0.0)

            pltpu.async_copy(ar, agg.at[iu2.at[ci]], ss, add=True)

        def wait_scatter(ci, p):
            ar, ss = bufs[p][0], bufs[p][6]
            pltpu.make_async_copy(ar, agg.at[iu2.at[ci]], ss).wait()

        issue(0, 0)

        @pl.loop(0, (NCHUNK - 1) // 2)
        def _(j):
            for p in range(2):
                ci = 2 * j + p
                wait_in(ci, p)
                @pl.when(ci > 0)
                def _():
                    wait_scatter(ci - 1, 1 - p)
                issue(ci + 1, 1 - p)
                compute_scatter(ci, p)

        ci_last = NCHUNK - 1
        wait_in(ci_last, ci_last % 2)
        wait_scatter(ci_last - 1, 1 - ci_last % 2)
        compute_scatter(ci_last, ci_last % 2)
        wait_scatter(ci_last, ci_last % 2)

        plsc.subcore_barrier()

        @pl.when(jnp.logical_and(cid == 0, sid < NZT))
        def _():
            pltpu.sync_copy(agg.at[pl.ds(r0, RPT)], out0.at[pl.ds(r0, RPT)])

        @pl.when(jnp.logical_and(cid == 1, sid < NZT))
        def _():
            pltpu.sync_copy(agg.at[pl.ds(r0, RPT)], out1.at[pl.ds(r0, RPT)])

    return body(a, b, c, iu.reshape(NW, NCHUNK, K), iv.reshape(NW, NCHUNK, K))


def _final_body(u_ref, p0_ref, p1_ref, wfu_ref, wfa_ref, bf_ref, wt_ref,
                bt_ref, o_ref):
    agg = p0_ref[...] + p1_ref[...]
    f = (
        jnp.dot(u_ref[...].astype(jnp.bfloat16), wfu_ref[...].astype(jnp.bfloat16),
                preferred_element_type=jnp.float32)
        + jnp.dot(agg.astype(jnp.bfloat16), wfa_ref[...].astype(jnp.bfloat16),
                  preferred_element_type=jnp.float32)
        + bf_ref[...]
    )
    f = jnp.maximum(f, 0.0)
    fb = f.astype(jnp.bfloat16).astype(jnp.float32)
    wtb = wt_ref[...].astype(jnp.bfloat16).astype(jnp.float32)
    t = jnp.sum(fb * wtb, axis=1, keepdims=True) + bt_ref[...]
    o_ref[...] = 1.0 / (1.0 + jnp.exp(-t))


def _tc_final(u, p0, p1, wfu, wfa, bf2, wt2, bt2):
    bu = 1000
    return pl.pallas_call(
        _final_body,
        grid=(U // bu,),
        in_specs=[
            pl.BlockSpec((bu, F), lambda i: (i, 0)),
            pl.BlockSpec((bu, D), lambda i: (i, 0)),
            pl.BlockSpec((bu, D), lambda i: (i, 0)),
            pl.BlockSpec((F, D), lambda i: (0, 0)),
            pl.BlockSpec((D, D), lambda i: (0, 0)),
            pl.BlockSpec((1, D), lambda i: (0, 0)),
            pl.BlockSpec((1, D), lambda i: (0, 0)),
            pl.BlockSpec((1, 1), lambda i: (0, 0)),
        ],
        out_specs=pl.BlockSpec((bu, 1), lambda i: (i, 0)),
        out_shape=jax.ShapeDtypeStruct((U, 1), jnp.float32),
    )(u, p0, p1, wfu, wfa, bf2, wt2, bt2)


def kernel(u, v, e_indices, e_values, Wg, bg, Wf, bf, Wt, bt):
    iv = e_indices[0].astype(jnp.int32)
    iu = e_indices[1].astype(jnp.int32)
    a, b = _tc_proj_uv(u, v, Wg[:F], Wg[F:2 * F], bg.reshape(1, D))
    c = _tc_proj_e(e_values, Wg[2 * F:])
    p0, p1 = _sc_agg(a, b, c, iu, iv)
    return _tc_final(u, p0, p1, Wf[:F], Wf[F:], bf.reshape(1, D),
                     Wt.reshape(1, D), bt.reshape(1, 1))


# speedup vs baseline: 5.0163x; 1.0349x over previous
import functools

import jax
import jax.numpy as jnp
from jax import lax
from jax.experimental import pallas as pl
from jax.experimental.pallas import tpu as pltpu
from jax.experimental.pallas import tpu_sc as plsc

U = 10000
V = 10000
E = 320000
F = 128
H = 16
D = 64

NC = 2
NS = 16
NW = NC * NS
EW = E // NW
K = 80
K2 = K // 2
EW2 = EW // 2
NCHUNK = EW // K
NZT = 10
RPT = U // NZT
ZR = 200


def _proj_uv_body(u_ref, v_ref, wu_ref, wv_ref, bg_ref, a_ref, b_ref):
    a_ref[...] = (
        jnp.dot(u_ref[...].astype(jnp.bfloat16), wu_ref[...].astype(jnp.bfloat16),
                preferred_element_type=jnp.float32)
        + bg_ref[...]
    )
    b_ref[...] = jnp.dot(v_ref[...].astype(jnp.bfloat16), wv_ref[...].astype(jnp.bfloat16),
                         preferred_element_type=jnp.float32)


def _tc_proj_uv(u, v, wu, wv, bg2):
    bu = 1000
    return pl.pallas_call(
        _proj_uv_body,
        grid=(U // bu,),
        in_specs=[
            pl.BlockSpec((bu, F), lambda i: (i, 0)),
            pl.BlockSpec((bu, F), lambda i: (i, 0)),
            pl.BlockSpec((F, D), lambda i: (0, 0)),
            pl.BlockSpec((F, D), lambda i: (0, 0)),
            pl.BlockSpec((1, D), lambda i: (0, 0)),
        ],
        out_specs=[
            pl.BlockSpec((bu, D), lambda i: (i, 0)),
            pl.BlockSpec((bu, D), lambda i: (i, 0)),
        ],
        out_shape=[
            jax.ShapeDtypeStruct((U, D), jnp.float32),
            jax.ShapeDtypeStruct((V, D), jnp.float32),
        ],
    )(u, v, wu, wv, bg2)


def _proj_e_body(e_ref, we_ref, c_ref):
    m = jnp.dot(e_ref[...].astype(jnp.bfloat16), we_ref[...],
                preferred_element_type=jnp.float32)
    c_ref[...] = m.reshape(m.shape[0] * 4, 2 * D)


def _tc_proj_e(e_values, we):
    pk = 8
    w8 = jax.scipy.linalg.block_diag(*([we.astype(jnp.bfloat16)] * pk))
    be = 4000
    return pl.pallas_call(
        _proj_e_body,
        grid=(E // pk // be,),
        in_specs=[
            pl.BlockSpec((be, pk * H), lambda i: (i, 0)),
            pl.BlockSpec((pk * H, pk * D), lambda i: (0, 0)),
        ],
        out_specs=pl.BlockSpec((4 * be, 2 * D), lambda i: (i, 0)),
        out_shape=jax.ShapeDtypeStruct((E // 2, 2 * D), jnp.float32),
    )(e_values.reshape(E // pk, pk * H), w8)


def _sc_agg(a, b, c, iu, iv):
    mesh = plsc.VectorSubcoreMesh(
        core_axis_name="c", subcore_axis_name="s", num_cores=NC, num_subcores=NS
    )

    @functools.partial(
        pl.kernel,
        out_type=(
            jax.ShapeDtypeStruct((U, D), jnp.float32),
            jax.ShapeDtypeStruct((U, D), jnp.float32),
        ),
        mesh=mesh,
        scratch_types=[
            pltpu.VMEM((NCHUNK, K), jnp.int32),
            pltpu.VMEM((NCHUNK, K), jnp.int32),
            pltpu.VMEM((K, D), jnp.float32),
            pltpu.VMEM((K, D), jnp.float32),
            pltpu.VMEM((K, D), jnp.float32),
            pltpu.VMEM((K, D), jnp.float32),
            pltpu.VMEM((ZR, D), jnp.float32),
            pltpu.VMEM_SHARED((U, D), jnp.float32),
            pltpu.SemaphoreType.DMA,
            pltpu.SemaphoreType.DMA,
            pltpu.SemaphoreType.DMA,
            pltpu.SemaphoreType.DMA,
            pltpu.SemaphoreType.DMA,
            pltpu.SemaphoreType.DMA,
            pltpu.SemaphoreType.DMA,
            pltpu.SemaphoreType.DMA,
        ],
        compiler_params=pltpu.CompilerParams(use_tc_tiling_on_sc=False),
    )
    def body(a_hbm, b_hbm, c_hbm, iu_hbm, iv_hbm, out0, out1,
             iu2, iv2, x0, x1, y0, y1, zr, agg,
             sa0, sb0, sc0, sa1, sb1, sc1, ss0, ss1):
        cid = lax.axis_index("c")
        sid = lax.axis_index("s")
        wid = cid * NS + sid
        r0 = sid * RPT
        bufs = ((x0, y0, sa0, sb0, sc0, ss0),
                (x1, y1, sa1, sb1, sc1, ss1))

        pltpu.sync_copy(iu_hbm.at[wid], iu2)
        pltpu.sync_copy(iv_hbm.at[wid], iv2)

        @pl.when(sid < NZT)
        def _():
            @pl.loop(0, ZR)
            def _(r):
                for q in range(D // 16):
                    zr[r, pl.ds(q * 16, 16)] = jnp.zeros((16,), jnp.float32)

            for t in range(RPT // ZR):
                pltpu.sync_copy(zr, agg.at[pl.ds(r0 + t * ZR, ZR)])

        plsc.subcore_barrier()

        def issue_c(ci, p):
            x, _, _, _, sc, _ = bufs[p]
            pltpu.async_copy(c_hbm.at[pl.ds(wid * EW + ci * K, K)], x, sc)

        def wait_c(ci, p):
            x, _, _, _, sc, _ = bufs[p]
            pltpu.make_async_copy(c_hbm.at[pl.ds(wid * EW + ci * K, K)], x, sc).wait()

        def issue_ab(ci, p):
            x, _, sa, sb, _, _ = bufs[p]
            pltpu.async_copy(a_hbm.at[iu2.at[ci]], x, sa, add=True)
            pltpu.async_copy(b_hbm.at[iv2.at[ci]], x, sb, add=True)

        def wait_ab(ci, p):
            x, _, sa, sb, _, _ = bufs[p]
            pltpu.make_async_copy(a_hbm.at[iu2.at[ci]], x, sa).wait()
            pltpu.make_async_copy(b_hbm.at[iv2.at[ci]], x, sb).wait()

        def relu(p):
            x, y = bufs[p][0], bufs[p][1]

            @pl.loop(0, K, unroll=8)
            def _(r):
                for q in range(D // 16):
                    sl = pl.ds(q * 16, 16)
                    y[r, sl] = jnp.maximum(x[r, sl], 0.0)

        def issue_scatter(ci, p):
            y, ss = bufs[p][1], bufs[p][5]
            pltpu.async_copy(y, agg.at[iu2.at[ci]], ss, add=True)

        def wait_scatter(ci, p):
            y, ss = bufs[p][1], bufs[p][5]
            pltpu.make_async_copy(y, agg.at[iu2.at[ci]], ss).wait()

        issue_c(0, 0)
        issue_c(1, 1)
        wait_c(0, 0)
        issue_ab(0, 0)

        @pl.loop(0, (NCHUNK - 1) // 2)
        def _(j):
            for p in range(2):
                ci = 2 * j + p
                wait_c(ci + 1, 1 - p)
                issue_ab(ci + 1, 1 - p)
                wait_ab(ci, p)

                @pl.when(ci >= 2)
                def _():
                    wait_scatter(ci - 2, p)

                relu(p)

                @pl.when(ci < NCHUNK - 2)
                def _():
                    issue_c(ci + 2, p)

                issue_scatter(ci, p)

        ci_last = NCHUNK - 1
        pl_ = ci_last % 2
        wait_ab(ci_last, pl_)
        wait_scatter(ci_last - 2, pl_)
        relu(pl_)
        issue_scatter(ci_last, pl_)
        wait_scatter(ci_last - 1, 1 - pl_)
        wait_scatter(ci_last, pl_)

        plsc.subcore_barrier()

        @pl.when(jnp.logical_and(cid == 0, sid < NZT))
        def _():
            pltpu.sync_copy(agg.at[pl.ds(r0, RPT)], out0.at[pl.ds(r0, RPT)])

        @pl.when(jnp.logical_and(cid == 1, sid < NZT))
        def _():
            pltpu.sync_copy(agg.at[pl.ds(r0, RPT)], out1.at[pl.ds(r0, RPT)])

    return body(a, b, c, iu.reshape(NW, NCHUNK, K), iv.reshape(NW, NCHUNK, K))


def _final_body(u_ref, p0_ref, p1_ref, wfu_ref, wfa_ref, bf_ref, wt_ref,
                bt_ref, o_ref):
    agg = p0_ref[...] + p1_ref[...]
    f = (
        jnp.dot(u_ref[...].astype(jnp.bfloat16), wfu_ref[...].astype(jnp.bfloat16),
                preferred_element_type=jnp.float32)
        + jnp.dot(agg.astype(jnp.bfloat16), wfa_ref[...].astype(jnp.bfloat16),
                  preferred_element_type=jnp.float32)
        + bf_ref[...]
    )
    f = jnp.maximum(f, 0.0)
    fb = f.astype(jnp.bfloat16).astype(jnp.float32)
    wtb = wt_ref[...].astype(jnp.bfloat16).astype(jnp.float32)
    t = jnp.sum(fb * wtb, axis=1, keepdims=True) + bt_ref[...]
    o_ref[...] = 1.0 / (1.0 + jnp.exp(-t))


def _tc_final(u, p0, p1, wfu, wfa, bf2, wt2, bt2):
    bu = 1000
    return pl.pallas_call(
        _final_body,
        grid=(U // bu,),
        in_specs=[
            pl.BlockSpec((bu, F), lambda i: (i, 0)),
            pl.BlockSpec((bu, D), lambda i: (i, 0)),
            pl.BlockSpec((bu, D), lambda i: (i, 0)),
            pl.BlockSpec((F, D), lambda i: (0, 0)),
            pl.BlockSpec((D, D), lambda i: (0, 0)),
            pl.BlockSpec((1, D), lambda i: (0, 0)),
            pl.BlockSpec((1, D), lambda i: (0, 0)),
            pl.BlockSpec((1, 1), lambda i: (0, 0)),
        ],
        out_specs=pl.BlockSpec((bu, 1), lambda i: (i, 0)),
        out_shape=jax.ShapeDtypeStruct((U, 1), jnp.float32),
    )(u, p0, p1, wfu, wfa, bf2, wt2, bt2)


def kernel(u, v, e_indices, e_values, Wg, bg, Wf, bf, Wt, bt):
    iv = e_indices[0].astype(jnp.int32)
    iu = e_indices[1].astype(jnp.int32)
    a, b = _tc_proj_uv(u, v, Wg[:F], Wg[F:2 * F], bg.reshape(1, D))
    c = _tc_proj_e(e_values, Wg[2 * F:]).reshape(E, D)
    p0, p1 = _sc_agg(a, b, c, iu, iv)
    return _tc_final(u, p0, p1, Wf[:F], Wf[F:], bf.reshape(1, D),
                     Wt.reshape(1, D), bt.reshape(1, 1))


# speedup vs baseline: 5.2512x; 1.0468x over previous
import functools

import jax
import jax.numpy as jnp
from jax import lax
from jax.experimental import pallas as pl
from jax.experimental.pallas import tpu as pltpu
from jax.experimental.pallas import tpu_sc as plsc

U = 10000
V = 10000
E = 320000
F = 128
H = 16
D = 64

NC = 2
NS = 16
NW = NC * NS
EW = E // NW
K = 80
NZT = 10
RPT = U // NZT
ZR = 200


def _proj_uv_body(u_ref, v_ref, wu_ref, wv_ref, bg_ref, a_ref, b_ref):
    a_ref[...] = (
        jnp.dot(u_ref[...].astype(jnp.bfloat16), wu_ref[...].astype(jnp.bfloat16),
                preferred_element_type=jnp.float32)
        + bg_ref[...]
    )
    b_ref[...] = jnp.dot(v_ref[...].astype(jnp.bfloat16), wv_ref[...].astype(jnp.bfloat16),
                         preferred_element_type=jnp.float32)


def _tc_proj_uv(u, v, wu, wv, bg2):
    bu = 1000
    return pl.pallas_call(
        _proj_uv_body,
        grid=(U // bu,),
        in_specs=[
            pl.BlockSpec((bu, F), lambda i: (i, 0)),
            pl.BlockSpec((bu, F), lambda i: (i, 0)),
            pl.BlockSpec((F, D), lambda i: (0, 0)),
            pl.BlockSpec((F, D), lambda i: (0, 0)),
            pl.BlockSpec((1, D), lambda i: (0, 0)),
        ],
        out_specs=[
            pl.BlockSpec((bu, D), lambda i: (i, 0)),
            pl.BlockSpec((bu, D), lambda i: (i, 0)),
        ],
        out_shape=[
            jax.ShapeDtypeStruct((U, D), jnp.float32),
            jax.ShapeDtypeStruct((V, D), jnp.float32),
        ],
    )(u, v, wu, wv, bg2)


def _proj_e_body(e_ref, we_ref, c_ref):
    m = jnp.dot(e_ref[...].astype(jnp.bfloat16), we_ref[...],
                preferred_element_type=jnp.float32)
    c_ref[...] = m.reshape(m.shape[0] * 4, 2 * D)


def _tc_proj_e(e_part, we, ne):
    pk = 8
    w8 = jax.scipy.linalg.block_diag(*([we.astype(jnp.bfloat16)] * pk))
    be = 4000
    return pl.pallas_call(
        _proj_e_body,
        grid=(ne // pk // be,),
        in_specs=[
            pl.BlockSpec((be, pk * H), lambda i: (i, 0)),
            pl.BlockSpec((pk * H, pk * D), lambda i: (0, 0)),
        ],
        out_specs=pl.BlockSpec((4 * be, 2 * D), lambda i: (i, 0)),
        out_shape=jax.ShapeDtypeStruct((ne // 2, 2 * D), jnp.float32),
    )(e_part.reshape(ne // pk, pk * H), w8)


def _sc_agg(a, b, c, iu, iv, e0, ew):
    ncb = ew // K
    mesh = plsc.VectorSubcoreMesh(
        core_axis_name="c", subcore_axis_name="s", num_cores=NC, num_subcores=NS
    )

    @functools.partial(
        pl.kernel,
        out_type=(
            jax.ShapeDtypeStruct((U, D), jnp.float32),
            jax.ShapeDtypeStruct((U, D), jnp.float32),
        ),
        mesh=mesh,
        scratch_types=[
            pltpu.VMEM((ncb, K), jnp.int32),
            pltpu.VMEM((ncb, K), jnp.int32),
            pltpu.VMEM((K, D), jnp.float32),
            pltpu.VMEM((K, D), jnp.float32),
            pltpu.VMEM((K, D), jnp.float32),
            pltpu.VMEM((K, D), jnp.float32),
            pltpu.VMEM((ZR, D), jnp.float32),
            pltpu.VMEM_SHARED((U, D), jnp.float32),
            pltpu.SemaphoreType.DMA,
            pltpu.SemaphoreType.DMA,
            pltpu.SemaphoreType.DMA,
            pltpu.SemaphoreType.DMA,
            pltpu.SemaphoreType.DMA,
            pltpu.SemaphoreType.DMA,
            pltpu.SemaphoreType.DMA,
            pltpu.SemaphoreType.DMA,
        ],
        compiler_params=pltpu.CompilerParams(use_tc_tiling_on_sc=False),
    )
    def body(a_hbm, b_hbm, c_hbm, iu_hbm, iv_hbm, out0, out1,
             iu2, iv2, x0, x1, y0, y1, zr, agg,
             sa0, sb0, sc0, sa1, sb1, sc1, ss0, ss1):
        cid = lax.axis_index("c")
        sid = lax.axis_index("s")
        wid = cid * NS + sid
        r0 = sid * RPT
        bufs = ((x0, y0, sa0, sb0, sc0, ss0),
                (x1, y1, sa1, sb1, sc1, ss1))

        pltpu.sync_copy(iu_hbm.at[wid], iu2)
        pltpu.sync_copy(iv_hbm.at[wid], iv2)

        @pl.when(sid < NZT)
        def _():
            @pl.loop(0, ZR)
            def _(r):
                for q in range(D // 16):
                    zr[r, pl.ds(q * 16, 16)] = jnp.zeros((16,), jnp.float32)

            for t in range(RPT // ZR):
                pltpu.sync_copy(zr, agg.at[pl.ds(r0 + t * ZR, ZR)])

        plsc.subcore_barrier()

        def issue_c(ci, p):
            x, _, _, _, sc, _ = bufs[p]
            pltpu.async_copy(c_hbm.at[pl.ds(wid * ew + ci * K, K)], x, sc)

        def wait_c(ci, p):
            x, _, _, _, sc, _ = bufs[p]
            pltpu.make_async_copy(
                c_hbm.at[pl.ds(wid * ew + ci * K, K)], x, sc).wait()

        def issue_ab(ci, p):
            x, _, sa, sb, _, _ = bufs[p]
            pltpu.async_copy(a_hbm.at[iu2.at[ci]], x, sa, add=True)
            pltpu.async_copy(b_hbm.at[iv2.at[ci]], x, sb, add=True)

        def wait_ab(ci, p):
            x, _, sa, sb, _, _ = bufs[p]
            pltpu.make_async_copy(a_hbm.at[iu2.at[ci]], x, sa).wait()
            pltpu.make_async_copy(b_hbm.at[iv2.at[ci]], x, sb).wait()

        def relu(p):
            x, y = bufs[p][0], bufs[p][1]

            @pl.loop(0, K, unroll=8)
            def _(r):
                for q in range(D // 16):
                    sl = pl.ds(q * 16, 16)
                    y[r, sl] = jnp.maximum(x[r, sl], 0.0)

        def issue_scatter(ci, p):
            y, ss = bufs[p][1], bufs[p][5]
            pltpu.async_copy(y, agg.at[iu2.at[ci]], ss, add=True)

        def wait_scatter(ci, p):
            y, ss = bufs[p][1], bufs[p][5]
            pltpu.make_async_copy(y, agg.at[iu2.at[ci]], ss).wait()

        issue_c(0, 0)
        issue_c(1, 1)
        wait_c(0, 0)
        issue_ab(0, 0)

        nmain = 2 * ((ncb - 1) // 2)

        @pl.loop(0, (ncb - 1) // 2)
        def _(j):
            for p in range(2):
                ci = 2 * j + p
                wait_c(ci + 1, 1 - p)
                issue_ab(ci + 1, 1 - p)
                wait_ab(ci, p)

                @pl.when(ci >= 2)
                def _():
                    wait_scatter(ci - 2, p)

                relu(p)

                @pl.when(ci < ncb - 2)
                def _():
                    issue_c(ci + 2, p)

                issue_scatter(ci, p)

        for ci in range(nmain, ncb):
            p = ci % 2
            if ci + 1 < ncb:
                wait_c(ci + 1, 1 - p)
                issue_ab(ci + 1, 1 - p)
            wait_ab(ci, p)
            if ci >= 2:
                wait_scatter(ci - 2, p)
            relu(p)
            issue_scatter(ci, p)

        wait_scatter(ncb - 2, (ncb - 2) % 2)
        wait_scatter(ncb - 1, (ncb - 1) % 2)

        plsc.subcore_barrier()

        @pl.when(jnp.logical_and(cid == 0, sid < NZT))
        def _():
            pltpu.sync_copy(agg.at[pl.ds(r0, RPT)], out0.at[pl.ds(r0, RPT)])

        @pl.when(jnp.logical_and(cid == 1, sid < NZT))
        def _():
            pltpu.sync_copy(agg.at[pl.ds(r0, RPT)], out1.at[pl.ds(r0, RPT)])

    sl = slice(e0, e0 + NW * ew)
    return body(a, b, c, iu[sl].reshape(NW, ncb, K), iv[sl].reshape(NW, ncb, K))


def _final_body(u_ref, p0_ref, p1_ref, p2_ref, p3_ref, wfu_ref, wfa_ref,
                bf_ref, wt_ref, bt_ref, o_ref):
    agg = (p0_ref[...] + p1_ref[...]) + (p2_ref[...] + p3_ref[...])
    f = (
        jnp.dot(u_ref[...].astype(jnp.bfloat16), wfu_ref[...].astype(jnp.bfloat16),
                preferred_element_type=jnp.float32)
        + jnp.dot(agg.astype(jnp.bfloat16), wfa_ref[...].astype(jnp.bfloat16),
                  preferred_element_type=jnp.float32)
        + bf_ref[...]
    )
    f = jnp.maximum(f, 0.0)
    fb = f.astype(jnp.bfloat16).astype(jnp.float32)
    wtb = wt_ref[...].astype(jnp.bfloat16).astype(jnp.float32)
    t = jnp.sum(fb * wtb, axis=1, keepdims=True) + bt_ref[...]
    o_ref[...] = 1.0 / (1.0 + jnp.exp(-t))


def _tc_final(u, parts, wfu, wfa, bf2, wt2, bt2):
    bu = 1000
    return pl.pallas_call(
        _final_body,
        grid=(U // bu,),
        in_specs=[
            pl.BlockSpec((bu, F), lambda i: (i, 0)),
            pl.BlockSpec((bu, D), lambda i: (i, 0)),
            pl.BlockSpec((bu, D), lambda i: (i, 0)),
            pl.BlockSpec((bu, D), lambda i: (i, 0)),
            pl.BlockSpec((bu, D), lambda i: (i, 0)),
            pl.BlockSpec((F, D), lambda i: (0, 0)),
            pl.BlockSpec((D, D), lambda i: (0, 0)),
            pl.BlockSpec((1, D), lambda i: (0, 0)),
            pl.BlockSpec((1, D), lambda i: (0, 0)),
            pl.BlockSpec((1, 1), lambda i: (0, 0)),
        ],
        out_specs=pl.BlockSpec((bu, 1), lambda i: (i, 0)),
        out_shape=jax.ShapeDtypeStruct((U, 1), jnp.float32),
    )(u, *parts, wfu, wfa, bf2, wt2, bt2)


E1 = 192000
EW1 = E1 // NW
EW2 = (E - E1) // NW


def kernel(u, v, e_indices, e_values, Wg, bg, Wf, bf, Wt, bt):
    iv = e_indices[0].astype(jnp.int32)
    iu = e_indices[1].astype(jnp.int32)
    a, b = _tc_proj_uv(u, v, Wg[:F], Wg[F:2 * F], bg.reshape(1, D))
    we = Wg[2 * F:]
    c1 = _tc_proj_e(e_values[:E1], we, E1).reshape(E1, D)
    c2 = _tc_proj_e(e_values[E1:], we, E - E1).reshape(E - E1, D)
    p0, p1 = _sc_agg(a, b, c1, iu, iv, 0, EW1)
    p2, p3 = _sc_agg(a, b, c2, iu, iv, E1, EW2)
    return _tc_final(u, (p0, p1, p2, p3), Wf[:F], Wf[F:], bf.reshape(1, D),
                     Wt.reshape(1, D), bt.reshape(1, 1))


# speedup vs baseline: 5.5694x; 1.0606x over previous
import functools

import jax
import jax.numpy as jnp
from jax import lax
from jax.experimental import pallas as pl
from jax.experimental.pallas import tpu as pltpu
from jax.experimental.pallas import tpu_sc as plsc

U = 10000
V = 10000
E = 320000
F = 128
H = 16
D = 64

NC = 2
NS = 16
NW = NC * NS
EW = E // NW
K = 80
NZT = 10
RPT = U // NZT
ZR = 200


def _proj_uv_body(u_ref, v_ref, wu_ref, wv_ref, bg_ref, a_ref, b_ref):
    a_ref[...] = (
        jnp.dot(u_ref[...].astype(jnp.bfloat16), wu_ref[...].astype(jnp.bfloat16),
                preferred_element_type=jnp.float32)
        + bg_ref[...]
    )
    b_ref[...] = jnp.dot(v_ref[...].astype(jnp.bfloat16), wv_ref[...].astype(jnp.bfloat16),
                         preferred_element_type=jnp.float32)


def _tc_proj_uv(u, v, wu, wv, bg2):
    bu = 1000
    return pl.pallas_call(
        _proj_uv_body,
        grid=(U // bu,),
        in_specs=[
            pl.BlockSpec((bu, F), lambda i: (i, 0)),
            pl.BlockSpec((bu, F), lambda i: (i, 0)),
            pl.BlockSpec((F, D), lambda i: (0, 0)),
            pl.BlockSpec((F, D), lambda i: (0, 0)),
            pl.BlockSpec((1, D), lambda i: (0, 0)),
        ],
        out_specs=[
            pl.BlockSpec((bu, D), lambda i: (i, 0)),
            pl.BlockSpec((bu, D), lambda i: (i, 0)),
        ],
        out_shape=[
            jax.ShapeDtypeStruct((U, D), jnp.float32),
            jax.ShapeDtypeStruct((V, D), jnp.float32),
        ],
    )(u, v, wu, wv, bg2)


def _proj_e_body(e_ref, we_ref, c_ref):
    m = jnp.dot(e_ref[...].astype(jnp.bfloat16), we_ref[...],
                preferred_element_type=jnp.float32)
    c_ref[...] = m.reshape(m.shape[0] * 4, 2 * D)


def _tc_proj_e(e_part, we, ne):
    pk = 8
    w8 = jax.scipy.linalg.block_diag(*([we.astype(jnp.bfloat16)] * pk))
    be = next(c for c in (4000, 3400, 3200, 2000, 1600, 1000, 800, 400, 200)
              if (ne // pk) % c == 0)
    return pl.pallas_call(
        _proj_e_body,
        grid=(ne // pk // be,),
        in_specs=[
            pl.BlockSpec((be, pk * H), lambda i: (i, 0)),
            pl.BlockSpec((pk * H, pk * D), lambda i: (0, 0)),
        ],
        out_specs=pl.BlockSpec((4 * be, 2 * D), lambda i: (i, 0)),
        out_shape=jax.ShapeDtypeStruct((ne // 2, 2 * D), jnp.float32),
    )(e_part.reshape(ne // pk, pk * H), w8)


def _sc_agg(a, b, c, iu, iv, e0, ew):
    ncb = ew // K
    mesh = plsc.VectorSubcoreMesh(
        core_axis_name="c", subcore_axis_name="s", num_cores=NC, num_subcores=NS
    )

    @functools.partial(
        pl.kernel,
        out_type=(
            jax.ShapeDtypeStruct((U, D), jnp.float32),
            jax.ShapeDtypeStruct((U, D), jnp.float32),
        ),
        mesh=mesh,
        scratch_types=[
            pltpu.VMEM((ncb, K), jnp.int32),
            pltpu.VMEM((ncb, K), jnp.int32),
            pltpu.VMEM((K, D), jnp.float32),
            pltpu.VMEM((K, D), jnp.float32),
            pltpu.VMEM((K, D), jnp.float32),
            pltpu.VMEM((K, D), jnp.float32),
            pltpu.VMEM((ZR, D), jnp.float32),
            pltpu.VMEM_SHARED((U, D), jnp.float32),
            pltpu.SemaphoreType.DMA,
            pltpu.SemaphoreType.DMA,
            pltpu.SemaphoreType.DMA,
            pltpu.SemaphoreType.DMA,
            pltpu.SemaphoreType.DMA,
            pltpu.SemaphoreType.DMA,
            pltpu.SemaphoreType.DMA,
            pltpu.SemaphoreType.DMA,
        ],
        compiler_params=pltpu.CompilerParams(use_tc_tiling_on_sc=False),
    )
    def body(a_hbm, b_hbm, c_hbm, iu_hbm, iv_hbm, out0, out1,
             iu2, iv2, x0, x1, y0, y1, zr, agg,
             sa0, sb0, sc0, sa1, sb1, sc1, ss0, ss1):
        cid = lax.axis_index("c")
        sid = lax.axis_index("s")
        wid = cid * NS + sid
        r0 = sid * RPT
        bufs = ((x0, y0, sa0, sb0, sc0, ss0),
                (x1, y1, sa1, sb1, sc1, ss1))

        pltpu.sync_copy(iu_hbm.at[wid], iu2)
        pltpu.sync_copy(iv_hbm.at[wid], iv2)

        @pl.when(sid < NZT)
        def _():
            @pl.loop(0, ZR)
            def _(r):
                for q in range(D // 16):
                    zr[r, pl.ds(q * 16, 16)] = jnp.zeros((16,), jnp.float32)

            for t in range(RPT // ZR):
                pltpu.sync_copy(zr, agg.at[pl.ds(r0 + t * ZR, ZR)])

        plsc.subcore_barrier()

        def issue_c(ci, p):
            x, _, _, _, sc, _ = bufs[p]
            pltpu.async_copy(c_hbm.at[pl.ds(wid * ew + ci * K, K)], x, sc)

        def wait_c(ci, p):
            x, _, _, _, sc, _ = bufs[p]
            pltpu.make_async_copy(
                c_hbm.at[pl.ds(wid * ew + ci * K, K)], x, sc).wait()

        def issue_ab(ci, p):
            x, _, sa, sb, _, _ = bufs[p]
            pltpu.async_copy(a_hbm.at[iu2.at[ci]], x, sa, add=True)
            pltpu.async_copy(b_hbm.at[iv2.at[ci]], x, sb, add=True)

        def wait_ab(ci, p):
            x, _, sa, sb, _, _ = bufs[p]
            pltpu.make_async_copy(a_hbm.at[iu2.at[ci]], x, sa).wait()
            pltpu.make_async_copy(b_hbm.at[iv2.at[ci]], x, sb).wait()

        def relu(p):
            x, y = bufs[p][0], bufs[p][1]

            @pl.loop(0, K, unroll=8)
            def _(r):
                for q in range(D // 16):
                    sl = pl.ds(q * 16, 16)
                    y[r, sl] = jnp.maximum(x[r, sl], 0.0)

        def issue_scatter(ci, p):
            y, ss = bufs[p][1], bufs[p][5]
            pltpu.async_copy(y, agg.at[iu2.at[ci]], ss, add=True)

        def wait_scatter(ci, p):
            y, ss = bufs[p][1], bufs[p][5]
            pltpu.make_async_copy(y, agg.at[iu2.at[ci]], ss).wait()

        issue_c(0, 0)
        issue_c(1, 1)
        wait_c(0, 0)
        issue_ab(0, 0)

        nmain = 2 * ((ncb - 1) // 2)

        @pl.loop(0, (ncb - 1) // 2)
        def _(j):
            for p in range(2):
                ci = 2 * j + p
                wait_c(ci + 1, 1 - p)
                issue_ab(ci + 1, 1 - p)
                wait_ab(ci, p)

                @pl.when(ci >= 2)
                def _():
                    wait_scatter(ci - 2, p)

                relu(p)

                @pl.when(ci < ncb - 2)
                def _():
                    issue_c(ci + 2, p)

                issue_scatter(ci, p)

        for ci in range(nmain, ncb):
            p = ci % 2
            if ci + 1 < ncb:
                wait_c(ci + 1, 1 - p)
                issue_ab(ci + 1, 1 - p)
            wait_ab(ci, p)
            if ci >= 2:
                wait_scatter(ci - 2, p)
            relu(p)
            issue_scatter(ci, p)

        wait_scatter(ncb - 2, (ncb - 2) % 2)
        wait_scatter(ncb - 1, (ncb - 1) % 2)

        plsc.subcore_barrier()

        @pl.when(jnp.logical_and(cid == 0, sid < NZT))
        def _():
            pltpu.sync_copy(agg.at[pl.ds(r0, RPT)], out0.at[pl.ds(r0, RPT)])

        @pl.when(jnp.logical_and(cid == 1, sid < NZT))
        def _():
            pltpu.sync_copy(agg.at[pl.ds(r0, RPT)], out1.at[pl.ds(r0, RPT)])

    sl = slice(e0, e0 + NW * ew)
    return body(a, b, c, iu[sl].reshape(NW, ncb, K), iv[sl].reshape(NW, ncb, K))


def _final_body(u_ref, p0_ref, p1_ref, p2_ref, p3_ref, wfu_ref, wfa_ref,
                bf_ref, wt_ref, bt_ref, o_ref):
    agg = (p0_ref[...] + p1_ref[...]) + (p2_ref[...] + p3_ref[...])
    f = (
        jnp.dot(u_ref[...].astype(jnp.bfloat16), wfu_ref[...].astype(jnp.bfloat16),
                preferred_element_type=jnp.float32)
        + jnp.dot(agg.astype(jnp.bfloat16), wfa_ref[...].astype(jnp.bfloat16),
                  preferred_element_type=jnp.float32)
        + bf_ref[...]
    )
    f = jnp.maximum(f, 0.0)
    fb = f.astype(jnp.bfloat16).astype(jnp.float32)
    wtb = wt_ref[...].astype(jnp.bfloat16).astype(jnp.float32)
    t = jnp.sum(fb * wtb, axis=1, keepdims=True) + bt_ref[...]
    o_ref[...] = 1.0 / (1.0 + jnp.exp(-t))


def _tc_final(u, parts, wfu, wfa, bf2, wt2, bt2):
    bu = 1000
    return pl.pallas_call(
        _final_body,
        grid=(U // bu,),
        in_specs=[
            pl.BlockSpec((bu, F), lambda i: (i, 0)),
            pl.BlockSpec((bu, D), lambda i: (i, 0)),
            pl.BlockSpec((bu, D), lambda i: (i, 0)),
            pl.BlockSpec((bu, D), lambda i: (i, 0)),
            pl.BlockSpec((bu, D), lambda i: (i, 0)),
            pl.BlockSpec((F, D), lambda i: (0, 0)),
            pl.BlockSpec((D, D), lambda i: (0, 0)),
            pl.BlockSpec((1, D), lambda i: (0, 0)),
            pl.BlockSpec((1, D), lambda i: (0, 0)),
            pl.BlockSpec((1, 1), lambda i: (0, 0)),
        ],
        out_specs=pl.BlockSpec((bu, 1), lambda i: (i, 0)),
        out_shape=jax.ShapeDtypeStruct((U, 1), jnp.float32),
    )(u, *parts, wfu, wfa, bf2, wt2, bt2)


E1 = 102400
EW1 = E1 // NW
EW2 = (E - E1) // NW


def kernel(u, v, e_indices, e_values, Wg, bg, Wf, bf, Wt, bt):
    iv = e_indices[0].astype(jnp.int32)
    iu = e_indices[1].astype(jnp.int32)
    a, b = _tc_proj_uv(u, v, Wg[:F], Wg[F:2 * F], bg.reshape(1, D))
    we = Wg[2 * F:]
    c1 = _tc_proj_e(e_values[:E1], we, E1).reshape(E1, D)
    c2 = _tc_proj_e(e_values[E1:], we, E - E1).reshape(E - E1, D)
    p0, p1 = _sc_agg(a, b, c1, iu, iv, 0, EW1)
    p2, p3 = _sc_agg(a, b, c2, iu, iv, E1, EW2)
    return _tc_final(u, (p0, p1, p2, p3), Wf[:F], Wf[F:], bf.reshape(1, D),
                     Wt.reshape(1, D), bt.reshape(1, 1))
